# Initial kernel scaffold; baseline (speedup 1.0000x reference)
import jax
import jax.numpy as jnp
from jax.experimental import pallas as pl


def kernel(x, edge_index, batch, params, eps, center):
    raise NotImplementedError("write your pallas kernel here")


# speedup vs baseline: 13.1725x; 13.1725x over previous
import functools

import jax
import jax.numpy as jnp
from jax import lax
from jax.experimental import pallas as pl
from jax.experimental.pallas import tpu as pltpu
from jax.experimental.pallas import tpu_sc as plsc

N = 10000
E = 320000
D = 128
H = 64
L = 3
G = 128

NC = 2
NS = 16
NW = NC * NS
EPW = E // NW
C = 80
CH = EPW // C
KF = 5
NGRP = CH // KF
N_PAD = 10240
ROWS_PER_TILE = N_PAD // NS


def _make_sc_segsum():
    mesh = plsc.VectorSubcoreMesh(core_axis_name="c", subcore_axis_name="s")

    @functools.partial(
        pl.kernel,
        out_type=jax.ShapeDtypeStruct((NC, N_PAD, H), jnp.float32),
        mesh=mesh,
        scratch_types=[
            pltpu.VMEM((CH, C), jnp.int32),
            pltpu.VMEM((CH, C), jnp.int32),
            pltpu.VMEM((KF, C, H), jnp.float32),
            pltpu.VMEM_SHARED((N_PAD, H), jnp.float32),
            pltpu.SemaphoreType.DMA,
        ],
        compiler_params=pltpu.CompilerParams(use_tc_tiling_on_sc=False),
    )
    def sc_segsum(y_hbm, src_hbm, dst_hbm, zero_hbm, out_hbm,
                  src_v, dst_v, rows_v, agg_sh, sem):
        c = lax.axis_index("c")
        s = lax.axis_index("s")
        wid = s * NC + c
        row0 = pl.multiple_of(s * ROWS_PER_TILE, 8)

        pltpu.sync_copy(src_hbm.at[wid], src_v)
        pltpu.sync_copy(dst_hbm.at[wid], dst_v)

        pltpu.sync_copy(zero_hbm.at[pl.ds(row0, ROWS_PER_TILE)],
                        agg_sh.at[pl.ds(row0, ROWS_PER_TILE)])
        plsc.subcore_barrier()

        def group(g, _):
            descs = []
            for b in range(KF):
                q = g * KF + b
                descs.append(pltpu.async_copy(
                    y_hbm.at[src_v.at[q]], rows_v.at[b], sem))
            for b in range(KF):
                descs[b].wait()
            for b in range(KF):
                q = g * KF + b
                pltpu.sync_copy(rows_v.at[b], agg_sh.at[dst_v.at[q]],
                                add=True)
            return 0

        lax.fori_loop(0, NGRP, group, 0)
        plsc.subcore_barrier()

        pltpu.sync_copy(agg_sh.at[pl.ds(row0, ROWS_PER_TILE)],
                        out_hbm.at[c, pl.ds(row0, ROWS_PER_TILE)])

    return sc_segsum


_sc_segsum = _make_sc_segsum()


def _mm_body(h_ref, w_ref, o_ref):
    o_ref[...] = jnp.dot(h_ref[...], w_ref[...],
                         preferred_element_type=jnp.float32)


def _layer_body(y_ref, agg_ref, scale_ref, b1_ref, w2_ref, b2_ref,
                wn_ref, batch_ref, ynext_ref, pooled_ref):
    t = (scale_ref[...] * y_ref[...] + agg_ref[0, :N] + agg_ref[1, :N]
         + b1_ref[...])
    u = jnp.maximum(t, 0.0)
    h = jnp.maximum(
        jnp.dot(u, w2_ref[...], preferred_element_type=jnp.float32)
        + b2_ref[...], 0.0)
    ynext_ref[...] = jnp.dot(h, wn_ref[...],
                             preferred_element_type=jnp.float32)
    oh = (lax.broadcasted_iota(jnp.int32, (G, N), 0)
          == batch_ref[...]).astype(jnp.float32)
    pooled_ref[...] = jnp.dot(oh, h, preferred_element_type=jnp.float32)


def _layer_last_body(y_ref, agg_ref, scale_ref, b1_ref, w2_ref, b2_ref,
                     batch_ref, pooled_ref):
    t = (scale_ref[...] * y_ref[...] + agg_ref[0, :N] + agg_ref[1, :N]
         + b1_ref[...])
    u = jnp.maximum(t, 0.0)
    h = jnp.maximum(
        jnp.dot(u, w2_ref[...], preferred_element_type=jnp.float32)
        + b2_ref[...], 0.0)
    oh = (lax.broadcasted_iota(jnp.int32, (G, N), 0)
          == batch_ref[...]).astype(jnp.float32)
    pooled_ref[...] = jnp.dot(oh, h, preferred_element_type=jnp.float32)


def kernel(x, edge_index, batch, params, eps, center):
    src3 = edge_index[0].reshape(NW, CH, C)
    dst3 = edge_index[1].reshape(NW, CH, C)
    zeros = jnp.zeros((N_PAD, H), jnp.float32)
    batch_row = batch.reshape(1, N)

    y = pl.pallas_call(
        _mm_body,
        out_shape=jax.ShapeDtypeStruct((N, H), jnp.float32),
    )(x, params[0][0])

    pooled = []
    for l in range(L):
        W1, b1, W2, b2 = params[l]
        agg = _sc_segsum(y, src3, dst3, zeros)
        scale = (1.0 + eps[l]).reshape(1, 1)
        if l + 1 < L:
            y, p = pl.pallas_call(
                _layer_body,
                out_shape=(jax.ShapeDtypeStruct((N, H), jnp.float32),
                           jax.ShapeDtypeStruct((G, H), jnp.float32)),
            )(y, agg, scale, b1.reshape(1, H), W2, b2.reshape(1, H),
              params[l + 1][0], batch_row)
        else:
            p = pl.pallas_call(
                _layer_last_body,
                out_shape=jax.ShapeDtypeStruct((G, H), jnp.float32),
            )(y, agg, scale, b1.reshape(1, H), W2, b2.reshape(1, H),
              batch_row)
        pooled.append(p)

    z = jnp.concatenate(pooled, axis=-1)
    return (z, center)


# speedup vs baseline: 18.6813x; 1.4182x over previous
import functools

import jax
import jax.numpy as jnp
from jax import lax
from jax.experimental import pallas as pl
from jax.experimental.pallas import tpu as pltpu
from jax.experimental.pallas import tpu_sc as plsc

N = 10000
E = 320000
D = 128
H = 64
L = 3
G = 128

NC = 2
NS = 16
NW = NC * NS
EPW = E // NW
C = 80
CH = EPW // C
KF = 5
NGRP = CH // KF
N_PAD = 10240
ROWS_PER_TILE = N_PAD // NS


def _make_sc_segsum():
    mesh = plsc.VectorSubcoreMesh(core_axis_name="c", subcore_axis_name="s")

    @functools.partial(
        pl.kernel,
        out_type=jax.ShapeDtypeStruct((NC, N_PAD, H), jnp.float32),
        mesh=mesh,
        scratch_types=[
            pltpu.VMEM((CH, C), jnp.int32),
            pltpu.VMEM((CH, C), jnp.int32),
            pltpu.VMEM((KF, C, H), jnp.float32),
            pltpu.VMEM_SHARED((N_PAD, H), jnp.float32),
            pltpu.SemaphoreType.DMA((KF,)),
        ],
        compiler_params=pltpu.CompilerParams(use_tc_tiling_on_sc=False),
    )
    def sc_segsum(y_hbm, src_hbm, dst_hbm, zero_hbm, out_hbm,
                  src_v, dst_v, rows_v, agg_sh, sem):
        c = lax.axis_index("c")
        s = lax.axis_index("s")
        wid = s * NC + c
        row0 = pl.multiple_of(s * ROWS_PER_TILE, 8)

        pltpu.sync_copy(src_hbm.at[wid], src_v)
        pltpu.sync_copy(dst_hbm.at[wid], dst_v)

        pltpu.sync_copy(zero_hbm.at[pl.ds(row0, ROWS_PER_TILE)],
                        agg_sh.at[pl.ds(row0, ROWS_PER_TILE)])
        plsc.subcore_barrier()

        def fire(g, b):
            pltpu.async_copy(y_hbm.at[src_v.at[g * KF + b]], rows_v.at[b],
                             sem.at[b])

        def drain(g, b):
            pltpu.make_async_copy(y_hbm.at[src_v.at[g * KF + b]],
                                  rows_v.at[b], sem.at[b]).wait()
            pltpu.sync_copy(rows_v.at[b], agg_sh.at[dst_v.at[g * KF + b]],
                            add=True)

        for b in range(KF):
            fire(0, b)

        def group(g, _):
            for b in range(KF):
                drain(g - 1, b)
                fire(g, b)
            return 0

        lax.fori_loop(1, NGRP, group, 0)
        for b in range(KF):
            drain(NGRP - 1, b)
        plsc.subcore_barrier()

        pltpu.sync_copy(agg_sh.at[pl.ds(row0, ROWS_PER_TILE)],
                        out_hbm.at[c, pl.ds(row0, ROWS_PER_TILE)])

    return sc_segsum


_sc_segsum = _make_sc_segsum()


def _mm_body(h_ref, w_ref, o_ref):
    o_ref[...] = jnp.dot(h_ref[...], w_ref[...],
                         preferred_element_type=jnp.float32)


def _layer_body(y_ref, agg_ref, scale_ref, b1_ref, w2_ref, b2_ref,
                wn_ref, batch_ref, ynext_ref, pooled_ref):
    t = (scale_ref[...] * y_ref[...] + agg_ref[0, :N] + agg_ref[1, :N]
         + b1_ref[...])
    u = jnp.maximum(t, 0.0)
    h = jnp.maximum(
        jnp.dot(u, w2_ref[...], preferred_element_type=jnp.float32)
        + b2_ref[...], 0.0)
    ynext_ref[...] = jnp.dot(h, wn_ref[...],
                             preferred_element_type=jnp.float32)
    oh = (lax.broadcasted_iota(jnp.int32, (G, N), 0)
          == batch_ref[...]).astype(jnp.float32)
    pooled_ref[...] = jnp.dot(oh, h, preferred_element_type=jnp.float32)


def _layer_last_body(y_ref, agg_ref, scale_ref, b1_ref, w2_ref, b2_ref,
                     batch_ref, pooled_ref):
    t = (scale_ref[...] * y_ref[...] + agg_ref[0, :N] + agg_ref[1, :N]
         + b1_ref[...])
    u = jnp.maximum(t, 0.0)
    h = jnp.maximum(
        jnp.dot(u, w2_ref[...], preferred_element_type=jnp.float32)
        + b2_ref[...], 0.0)
    oh = (lax.broadcasted_iota(jnp.int32, (G, N), 0)
          == batch_ref[...]).astype(jnp.float32)
    pooled_ref[...] = jnp.dot(oh, h, preferred_element_type=jnp.float32)


def kernel(x, edge_index, batch, params, eps, center):
    src3 = edge_index[0].reshape(NW, CH, C)
    dst3 = edge_index[1].reshape(NW, CH, C)
    zeros = jnp.zeros((N_PAD, H), jnp.float32)
    batch_row = batch.reshape(1, N)

    y = pl.pallas_call(
        _mm_body,
        out_shape=jax.ShapeDtypeStruct((N, H), jnp.float32),
    )(x, params[0][0])

    pooled = []
    for l in range(L):
        W1, b1, W2, b2 = params[l]
        agg = _sc_segsum(y, src3, dst3, zeros)
        scale = (1.0 + eps[l]).reshape(1, 1)
        if l + 1 < L:
            y, p = pl.pallas_call(
                _layer_body,
                out_shape=(jax.ShapeDtypeStruct((N, H), jnp.float32),
                           jax.ShapeDtypeStruct((G, H), jnp.float32)),
            )(y, agg, scale, b1.reshape(1, H), W2, b2.reshape(1, H),
              params[l + 1][0], batch_row)
        else:
            p = pl.pallas_call(
                _layer_last_body,
                out_shape=jax.ShapeDtypeStruct((G, H), jnp.float32),
            )(y, agg, scale, b1.reshape(1, H), W2, b2.reshape(1, H),
              batch_row)
        pooled.append(p)

    z = jnp.concatenate(pooled, axis=-1)
    return (z, center)
